# baseline (speedup 1.0000x reference)
import jax
import jax.numpy as jnp
from jax.experimental import pallas as pl
from jax.experimental.pallas import tpu as pltpu
from jax.experimental.pallas import tpu_sc as plsc

_B = 2
_S = 2048
_H = 2048
_E = 8
_A = 128
_T = _B * _S
_TM = 512
_SC_BLK = 128
_SC_V = 16


def _router_kernel(xr_ref, wr_ref, logits_ref, logits_t_ref):
    xr = xr_ref[...].astype(jnp.bfloat16)
    wr = wr_ref[...]
    logits_ref[...] = jax.lax.dot_general(
        xr, wr, dimension_numbers=(((1,), (1,)), ((), ())),
        preferred_element_type=jnp.float32)
    logits_t_ref[...] = jax.lax.dot_general(
        wr, xr, dimension_numbers=(((1,), (1,)), ((), ())),
        preferred_element_type=jnp.float32)


def _sc_routing(logits_t):
    mesh = plsc.VectorSubcoreMesh(core_axis_name="core",
                                  subcore_axis_name="subcore")

    @pl.kernel(out_type=jax.ShapeDtypeStruct((_E, _T), jnp.float32),
               mesh=mesh)
    def k(l_hbm, w_hbm):
        def body(l_vmem, w_vmem):
            @pl.loop(0, _SC_BLK, step=_SC_V)
            def _(c):
                sl = pl.ds(c, _SC_V)
                v = [l_vmem[e, sl] for e in range(_E)]
                one = v[0] * 0.0 + 1.0
                zero = v[0] * 0.0
                m1 = v[0]
                for e in range(1, _E):
                    m1 = jnp.maximum(m1, v[e])
                sel1 = []
                seen = zero
                for e in range(_E):
                    hit = jnp.where(v[e] == m1, one, zero) * (one - seen)
                    seen = seen + hit
                    sel1.append(hit)
                vm = [v[e] - sel1[e] * 1e30 for e in range(_E)]
                m2 = vm[0]
                for e in range(1, _E):
                    m2 = jnp.maximum(m2, vm[e])
                sel2 = []
                seen2 = zero
                for e in range(_E):
                    hit = jnp.where(vm[e] == m2, one, zero) * (one - seen2)
                    seen2 = seen2 + hit
                    sel2.append(hit)
                t = jnp.exp(m2 - m1)
                w1 = 1.0 / (1.0 + t)
                w2 = t * w1
                for e in range(_E):
                    w_vmem[e, sl] = sel1[e] * w1 + sel2[e] * w2

        pltpu.emit_pipeline(
            body,
            grid=(_T // _SC_BLK,),
            in_specs=[pl.BlockSpec((_E, _SC_BLK), lambda i: (0, i))],
            out_specs=[pl.BlockSpec((_E, _SC_BLK), lambda i: (0, i))],
            core_axis_name=("core", "subcore"),
            dimension_semantics=(pltpu.PARALLEL,),
        )(l_hbm, w_hbm)

    return k(logits_t)


def _main_kernel(xi_ref, xo_ref, wt_ref, wd_ref, wu_ref, out_ref):
    lane = jax.lax.broadcasted_iota(jnp.int32, (_E, _E * _A), 1)
    row = jax.lax.broadcasted_iota(jnp.int32, (_E, _E * _A), 0)
    blockmat = (lane // _A == row).astype(jnp.bfloat16)
    scale = jax.lax.dot_general(
        wt_ref[...].astype(jnp.bfloat16), blockmat,
        dimension_numbers=(((0,), (0,)), ((), ())),
        preferred_element_type=jnp.float32)

    h = jax.lax.dot_general(
        xi_ref[...].astype(jnp.bfloat16), wd_ref[...],
        dimension_numbers=(((1,), (1,)), ((), ())),
        preferred_element_type=jnp.float32)
    h = jax.nn.gelu(h)
    h = (h * scale).astype(jnp.bfloat16)
    out = jax.lax.dot_general(
        h, wu_ref[...],
        dimension_numbers=(((1,), (0,)), ((), ())),
        preferred_element_type=jnp.float32)
    out_ref[...] = out + xo_ref[...]


def kernel(input_hidden_states, output_hidden_states, router_hidden_states,
           W_router, W_down, W_up):
    orig_shape = output_hidden_states.shape
    xi = input_hidden_states.reshape(_T, _H)
    xo = output_hidden_states.reshape(_T, _H)
    xr = router_hidden_states.reshape(_T, _H)
    wd = W_down.reshape(_E * _A, _H).astype(jnp.bfloat16)
    wu = W_up.transpose(0, 2, 1).reshape(_E * _A, _H).astype(jnp.bfloat16)

    grid = (_T // _TM,)
    logits, logits_t = pl.pallas_call(
        _router_kernel,
        grid=grid,
        in_specs=[
            pl.BlockSpec((_TM, _H), lambda i: (i, 0)),
            pl.BlockSpec((_E, _H), lambda i: (0, 0)),
        ],
        out_specs=[
            pl.BlockSpec((_TM, _E), lambda i: (i, 0)),
            pl.BlockSpec((_E, _TM), lambda i: (0, i)),
        ],
        out_shape=[
            jax.ShapeDtypeStruct((_T, _E), jnp.float32),
            jax.ShapeDtypeStruct((_E, _T), jnp.float32),
        ],
        compiler_params=pltpu.CompilerParams(
            dimension_semantics=("arbitrary",),
        ),
    )(xr, W_router)

    w_t = _sc_routing(logits_t)

    out = pl.pallas_call(
        _main_kernel,
        grid=grid,
        in_specs=[
            pl.BlockSpec((_TM, _H), lambda i: (i, 0)),
            pl.BlockSpec((_TM, _H), lambda i: (i, 0)),
            pl.BlockSpec((_E, _TM), lambda i: (0, i)),
            pl.BlockSpec((_E * _A, _H), lambda i: (0, 0)),
            pl.BlockSpec((_E * _A, _H), lambda i: (0, 0)),
        ],
        out_specs=pl.BlockSpec((_TM, _H), lambda i: (i, 0)),
        out_shape=jax.ShapeDtypeStruct((_T, _H), jnp.float32),
        compiler_params=pltpu.CompilerParams(
            dimension_semantics=("arbitrary",),
        ),
    )(xi, xo, w_t, wd, wu)

    return out.reshape(orig_shape), logits


# speedup vs baseline: 1.3047x; 1.3047x over previous
import jax
import jax.numpy as jnp
from jax.experimental import pallas as pl
from jax.experimental.pallas import tpu as pltpu

_B = 2
_S = 2048
_H = 2048
_E = 8
_A = 128
_T = _B * _S
_TM = 512


def _moe_tile_kernel(xi_ref, xo_ref, xr_ref, wr_ref, wd_ref, wu_ref,
                     out_ref, logits_ref, wd_bf):
    @pl.when(pl.program_id(0) == 0)
    def _():
        wd_bf[...] = wd_ref[...].astype(jnp.bfloat16)

    xr = xr_ref[...].astype(jnp.bfloat16)
    logits = jax.lax.dot_general(
        xr, wr_ref[...],
        dimension_numbers=(((1,), (1,)), ((), ())),
        preferred_element_type=jnp.float32)
    logits_ref[...] = logits

    p = jax.nn.softmax(logits, axis=-1)
    idx = jax.lax.broadcasted_iota(jnp.int32, (_TM, _E), 1)
    p1 = jnp.max(p, axis=-1, keepdims=True)
    i1 = jnp.min(jnp.where(p == p1, idx, _E), axis=-1, keepdims=True)
    sel1 = idx == i1
    pm = jnp.where(sel1, -jnp.inf, p)
    p2 = jnp.max(pm, axis=-1, keepdims=True)
    i2 = jnp.min(jnp.where(pm == p2, idx, _E), axis=-1, keepdims=True)
    sel2 = idx == i2
    denom = p1 + p2
    w = (jnp.where(sel1, p, 0.0) + jnp.where(sel2, p, 0.0)) / denom

    lane = jax.lax.broadcasted_iota(jnp.int32, (_E, _E * _A), 1)
    row = jax.lax.broadcasted_iota(jnp.int32, (_E, _E * _A), 0)
    blockmat = (lane // _A == row).astype(jnp.bfloat16)
    scale = jax.lax.dot_general(
        w.astype(jnp.bfloat16), blockmat,
        dimension_numbers=(((1,), (0,)), ((), ())),
        preferred_element_type=jnp.float32)

    h = jax.lax.dot_general(
        xi_ref[...].astype(jnp.bfloat16), wd_bf[...],
        dimension_numbers=(((1,), (1,)), ((), ())),
        preferred_element_type=jnp.float32)
    h = jax.nn.gelu(h)
    h = (h * scale).astype(jnp.bfloat16)
    out = jax.lax.dot_general(
        h, wu_ref[...],
        dimension_numbers=(((1,), (0,)), ((), ())),
        preferred_element_type=jnp.float32)
    out_ref[...] = out + xo_ref[...]


def kernel(input_hidden_states, output_hidden_states, router_hidden_states,
           W_router, W_down, W_up):
    orig_shape = output_hidden_states.shape
    xi = input_hidden_states.reshape(_T, _H)
    xo = output_hidden_states.reshape(_T, _H)
    xr = router_hidden_states.reshape(_T, _H)
    wd = W_down.reshape(_E * _A, _H)
    wu = W_up.transpose(0, 2, 1).reshape(_E * _A, _H).astype(jnp.bfloat16)

    grid = (_T // _TM,)
    out, logits = pl.pallas_call(
        _moe_tile_kernel,
        grid=grid,
        in_specs=[
            pl.BlockSpec((_TM, _H), lambda i: (i, 0)),
            pl.BlockSpec((_TM, _H), lambda i: (i, 0)),
            pl.BlockSpec((_TM, _H), lambda i: (i, 0)),
            pl.BlockSpec((_E, _H), lambda i: (0, 0)),
            pl.BlockSpec((_E * _A, _H), lambda i: (0, 0)),
            pl.BlockSpec((_E * _A, _H), lambda i: (0, 0)),
        ],
        out_specs=[
            pl.BlockSpec((_TM, _H), lambda i: (i, 0)),
            pl.BlockSpec((_TM, _E), lambda i: (i, 0)),
        ],
        out_shape=[
            jax.ShapeDtypeStruct((_T, _H), jnp.float32),
            jax.ShapeDtypeStruct((_T, _E), jnp.float32),
        ],
        scratch_shapes=[pltpu.VMEM((_E * _A, _H), jnp.bfloat16)],
        compiler_params=pltpu.CompilerParams(
            dimension_semantics=("arbitrary",),
        ),
    )(xi, xo, xr, W_router, wd, wu)

    return out.reshape(orig_shape), logits
